# Initial kernel scaffold; baseline (speedup 1.0000x reference)
import jax
import jax.numpy as jnp
from jax.experimental import pallas as pl


def kernel(features, table, feature_offsets):
    raise NotImplementedError("write your pallas kernel here")


# speedup vs baseline: 3.5204x; 3.5204x over previous
import functools

import jax
import jax.numpy as jnp
from jax import lax
from jax.experimental import pallas as pl
from jax.experimental.pallas import tpu as pltpu
from jax.experimental.pallas import tpu_sc as plsc

B = 4096
F = 100
D = 128
TOT = B * F
NC, NS, L = 2, 16, 16
NW = NC * NS
PER_W = TOT // NW
CHUNK = 128
NBUF = 4
NCHUNK = PER_W // CHUNK
NGROUP = NCHUNK // NBUF
NVEC = PER_W // L
OFF_TILED = 4 * F


def _sc_gather(features_flat, table, feature_offsets):
  mesh = plsc.VectorSubcoreMesh(core_axis_name="c", subcore_axis_name="s")

  @functools.partial(
      pl.kernel,
      out_type=jax.ShapeDtypeStruct((TOT, D), jnp.float32),
      mesh=mesh,
      scratch_types=[
          pltpu.VMEM((PER_W,), jnp.int32),
          pltpu.VMEM((OFF_TILED,), jnp.int32),
          [pltpu.VMEM((CHUNK, D), jnp.float32) for _ in range(NBUF)],
          [pltpu.SemaphoreType.DMA for _ in range(NBUF)],
          [pltpu.SemaphoreType.DMA for _ in range(NBUF)],
      ],
  )
  def k(feat_hbm, table_hbm, off_hbm, out_hbm, idx_v, off_v, bufs, gsems, wsems):
    wid = lax.axis_index("s") * NC + lax.axis_index("c")
    base = wid * PER_W

    pltpu.sync_copy(feat_hbm.at[pl.ds(base, PER_W)], idx_v)
    pltpu.sync_copy(off_hbm, off_v)

    def add_body(j, _):
      ph = lax.rem(j, OFF_TILED // L) * L
      s = pl.ds(j * L, L)
      idx_v[s] = idx_v[s] + off_v[pl.ds(ph, L)]
      return 0

    lax.fori_loop(0, NVEC, add_body, 0, unroll=4)

    def group_body(g, _):
      gh = []
      for b in range(NBUF):
        c = g * NBUF + b
        h = pltpu.async_copy(
            table_hbm.at[idx_v.at[pl.ds(c * CHUNK, CHUNK)]], bufs[b], gsems[b])
        gh.append(h)
      wh = []
      for b in range(NBUF):
        c = g * NBUF + b
        gh[b].wait()
        wh.append(pltpu.async_copy(
            bufs[b], out_hbm.at[pl.ds(base + c * CHUNK, CHUNK)], wsems[b]))
      for b in range(NBUF):
        wh[b].wait()
      return 0

    lax.fori_loop(0, NGROUP, group_body, 0)

  return k(features_flat, table, feature_offsets)


def kernel(features, table, feature_offsets):
  off_tiled = jnp.tile(feature_offsets, OFF_TILED // F)
  out = _sc_gather(features.reshape(TOT), table, off_tiled)
  return out.reshape(B, F, D)


# speedup vs baseline: 3.5688x; 1.0138x over previous
import functools

import jax
import jax.numpy as jnp
from jax import lax
from jax.experimental import pallas as pl
from jax.experimental.pallas import tpu as pltpu
from jax.experimental.pallas import tpu_sc as plsc

B = 4096
F = 100
D = 128
TOT = B * F
NC, NS, L = 2, 16, 16
NW = NC * NS
PER_W = TOT // NW
CHUNK = 128
NBUF = 4
NCHUNK = PER_W // CHUNK
NGROUP = NCHUNK // NBUF
NVEC = PER_W // L
OFF_TILED = 4 * F


def _sc_gather(features_flat, table, feature_offsets):
  mesh = plsc.VectorSubcoreMesh(core_axis_name="c", subcore_axis_name="s")

  @functools.partial(
      pl.kernel,
      out_type=jax.ShapeDtypeStruct((TOT, D), jnp.float32),
      mesh=mesh,
      scratch_types=[
          pltpu.VMEM((PER_W,), jnp.int32),
          pltpu.VMEM((OFF_TILED,), jnp.int32),
          [pltpu.VMEM((CHUNK, D), jnp.float32) for _ in range(NBUF)],
          [pltpu.SemaphoreType.DMA for _ in range(NBUF)],
          [pltpu.SemaphoreType.DMA for _ in range(NBUF)],
      ],
  )
  def k(feat_hbm, table_hbm, off_hbm, out_hbm, idx_v, off_v, bufs, gsems, wsems):
    wid = lax.axis_index("s") * NC + lax.axis_index("c")
    base = wid * PER_W

    pltpu.sync_copy(feat_hbm.at[pl.ds(base, PER_W)], idx_v)
    pltpu.sync_copy(off_hbm, off_v)

    def add_body(j, _):
      ph = lax.rem(j, OFF_TILED // L) * L
      s = pl.ds(j * L, L)
      idx_v[s] = idx_v[s] + off_v[pl.ds(ph, L)]
      return 0

    lax.fori_loop(0, NVEC, add_body, 0, unroll=4)

    def gather_desc(c, b):
      return pltpu.make_async_copy(
          table_hbm.at[idx_v.at[pl.ds(c * CHUNK, CHUNK)]], bufs[b], gsems[b])

    for b in range(NBUF):
      gather_desc(b, b).start()

    def group_body(g, _):
      for b in range(NBUF):
        c = g * NBUF + b
        gather_desc(c, b).wait()
        pltpu.async_copy(
            bufs[b], out_hbm.at[pl.ds(base + c * CHUNK, CHUNK)],
            wsems[b]).wait()

        @pl.when(c + NBUF < NCHUNK)
        def _():
          gather_desc(c + NBUF, b).start()

      return 0

    lax.fori_loop(0, NGROUP, group_body, 0)

  return k(features_flat, table, feature_offsets)


def kernel(features, table, feature_offsets):
  off_tiled = jnp.tile(feature_offsets, OFF_TILED // F)
  out = _sc_gather(features.reshape(TOT), table, off_tiled)
  return out.reshape(B, F, D)


# speedup vs baseline: 10.6752x; 2.9912x over previous
import functools

import jax
import jax.numpy as jnp
from jax import lax
from jax.experimental import pallas as pl
from jax.experimental.pallas import tpu as pltpu
from jax.experimental.pallas import tpu_sc as plsc

B = 4096
F = 100
D = 128
TOT = B * F
NC, NS, L = 2, 16, 16
NW = NC * NS
PER_W = TOT // NW
CHUNK = 128
NBUF = 5
LOOKAHEAD = 3
NCHUNK = PER_W // CHUNK
NVEC = PER_W // L
BLK_PER_F = B // L


def _sc_gather(features_fmajor, table, off_rep):
  mesh = plsc.VectorSubcoreMesh(core_axis_name="c", subcore_axis_name="s")

  @functools.partial(
      pl.kernel,
      out_type=jax.ShapeDtypeStruct((TOT, D), jnp.float32),
      mesh=mesh,
      scratch_types=[
          pltpu.VMEM((PER_W,), jnp.int32),
          pltpu.VMEM((F * L,), jnp.int32),
          [pltpu.VMEM((CHUNK, D), jnp.float32) for _ in range(NBUF)],
          [pltpu.SemaphoreType.DMA for _ in range(NBUF)],
          [pltpu.SemaphoreType.DMA for _ in range(NBUF)],
      ],
  )
  def k(feat_hbm, table_hbm, off_hbm, out_hbm, idx_v, off_v, bufs, gsems, wsems):
    wid = lax.axis_index("s") * NC + lax.axis_index("c")
    base = wid * PER_W

    pltpu.sync_copy(feat_hbm.at[pl.ds(base, PER_W)], idx_v)
    pltpu.sync_copy(off_hbm, off_v)

    blk0 = wid * NVEC

    def add_body(j, _):
      fld = (blk0 + j) // BLK_PER_F
      s = pl.ds(j * L, L)
      idx_v[s] = idx_v[s] + off_v[pl.ds(fld * L, L)]
      return 0

    lax.fori_loop(0, NVEC, add_body, 0, unroll=4)

    def gather_desc(c, b):
      return pltpu.make_async_copy(
          table_hbm.at[idx_v.at[pl.ds(c * CHUNK, CHUNK)]], bufs[b], gsems[b])

    def wb_desc(c, b):
      return pltpu.make_async_copy(
          bufs[b], out_hbm.at[pl.ds(base + c * CHUNK, CHUNK)], wsems[b])

    for c in range(LOOKAHEAD):
      gather_desc(c, c % NBUF).start()

    def group_body(g, _):
      for b in range(NBUF):
        c = g * NBUF + b
        ca = c + LOOKAHEAD
        ba = (b + LOOKAHEAD) % NBUF
        cd = ca - NBUF

        @pl.when((ca < NCHUNK) & (cd >= 0))
        def _():
          wb_desc(cd, ba).wait()

        @pl.when(ca < NCHUNK)
        def _():
          gather_desc(ca, ba).start()

        gather_desc(c, b).wait()
        wb_desc(c, b).start()
      return 0

    lax.fori_loop(0, NCHUNK // NBUF, group_body, 0)

    for cc in range(NCHUNK - NBUF, NCHUNK):
      wb_desc(cc, cc % NBUF).wait()

  return k(features_fmajor, table, off_rep)


def kernel(features, table, feature_offsets):
  feats_fmajor = jnp.transpose(features).reshape(TOT)
  off_rep = jnp.repeat(feature_offsets, L)
  out = _sc_gather(feats_fmajor, table, off_rep)
  return out.reshape(F, B, D).transpose(1, 0, 2)


# speedup vs baseline: 10.8991x; 1.0210x over previous
import functools

import jax
import jax.numpy as jnp
from jax import lax
from jax.experimental import pallas as pl
from jax.experimental.pallas import tpu as pltpu
from jax.experimental.pallas import tpu_sc as plsc

B = 4096
F = 100
D = 128
TOT = B * F
NC, NS, L = 2, 16, 16
NW = NC * NS
PER_W = TOT // NW
CHUNK = 128
NBUF = 5
LOOKAHEAD = 3
NCHUNK = PER_W // CHUNK
NVEC = PER_W // L
BLK_PER_F = B // L


def _sc_gather(features_fmajor, table, off_rep):
  mesh = plsc.VectorSubcoreMesh(core_axis_name="c", subcore_axis_name="s")

  @functools.partial(
      pl.kernel,
      out_type=jax.ShapeDtypeStruct((TOT, D), jnp.float32),
      mesh=mesh,
      scratch_types=[
          pltpu.VMEM((PER_W,), jnp.int32),
          pltpu.VMEM((F * L,), jnp.int32),
          [pltpu.VMEM((CHUNK, D), jnp.float32) for _ in range(NBUF)],
          [pltpu.SemaphoreType.DMA for _ in range(NBUF)],
          [pltpu.SemaphoreType.DMA for _ in range(NBUF)],
      ],
  )
  def k(feat_hbm, table_hbm, off_hbm, out_hbm, idx_v, off_v, bufs, gsems, wsems):
    wid = lax.axis_index("s") * NC + lax.axis_index("c")
    base = wid * PER_W

    pltpu.sync_copy(feat_hbm.at[pl.ds(base, PER_W)], idx_v)
    pltpu.sync_copy(off_hbm, off_v)

    blk0 = wid * NVEC
    BLK_PER_C = CHUNK // L

    def add_chunk(c):
      for t in range(BLK_PER_C):
        j = c * BLK_PER_C + t
        fld = (blk0 + j) // BLK_PER_F
        s = pl.ds(j * L, L)
        idx_v[s] = idx_v[s] + off_v[pl.ds(fld * L, L)]

    def gather_desc(c, b):
      return pltpu.make_async_copy(
          table_hbm.at[idx_v.at[pl.ds(c * CHUNK, CHUNK)]], bufs[b], gsems[b])

    def wb_desc(c, b):
      return pltpu.make_async_copy(
          bufs[b], out_hbm.at[pl.ds(base + c * CHUNK, CHUNK)], wsems[b])

    for c in range(LOOKAHEAD):
      add_chunk(c)
      gather_desc(c, c % NBUF).start()

    def group_body(g, _):
      for b in range(NBUF):
        c = g * NBUF + b
        ca = c + LOOKAHEAD
        ba = (b + LOOKAHEAD) % NBUF
        cd = ca - NBUF

        @pl.when((ca < NCHUNK) & (cd >= 0))
        def _():
          wb_desc(cd, ba).wait()

        @pl.when(ca < NCHUNK)
        def _():
          add_chunk(ca)
          gather_desc(ca, ba).start()

        gather_desc(c, b).wait()
        wb_desc(c, b).start()
      return 0

    lax.fori_loop(0, NCHUNK // NBUF, group_body, 0)

    for cc in range(NCHUNK - NBUF, NCHUNK):
      wb_desc(cc, cc % NBUF).wait()

  return k(features_fmajor, table, off_rep)


def kernel(features, table, feature_offsets):
  feats_fmajor = jnp.transpose(features).reshape(TOT)
  off_rep = jnp.repeat(feature_offsets, L)
  out = _sc_gather(feats_fmajor, table, off_rep)
  return out.reshape(F, B, D).transpose(1, 0, 2)
